# baseline (speedup 1.0000x reference)
import functools

import jax
import jax.numpy as jnp
from jax import lax
from jax.experimental import pallas as pl
from jax.experimental.pallas import tpu as pltpu
from jax.experimental.pallas import tpu_sc as plsc

N = 10000
D = 128
H = 16
NC = 2
NS = 16
NW = NC * NS
CHUNK = 512
CPT = 20
NB = 4
EPT = CPT * CHUNK
EPAD = NW * EPT
RPS = 632
NPAD = NS * RPS


def _sc_scatter(y, src_t, dst_t):
    mesh = plsc.VectorSubcoreMesh(core_axis_name="c", subcore_axis_name="s")

    @functools.partial(
        pl.kernel,
        out_type=jax.ShapeDtypeStruct((NC, NPAD, H), jnp.float32),
        mesh=mesh,
        scratch_types=[
            pltpu.VMEM((CPT, CHUNK), jnp.int32),
            pltpu.VMEM((CPT, CHUNK), jnp.int32),
            pltpu.VMEM((NB, CHUNK, H), jnp.float32),
            pltpu.VMEM((RPS, H), jnp.float32),
            pltpu.VMEM_SHARED((NPAD, H), jnp.float32),
            pltpu.SemaphoreType.DMA,
        ],
        compiler_params=pltpu.CompilerParams(use_tc_tiling_on_sc=False),
    )
    def k(y_hbm, src_hbm, dst_hbm, out_hbm, src_v, dst_v, rows_v, zero_v,
          agg_sh, sem):
        cid = lax.axis_index("c")
        sid = lax.axis_index("s")
        wid = cid * NS + sid

        pltpu.sync_copy(src_hbm.at[wid], src_v)
        pltpu.sync_copy(dst_hbm.at[wid], dst_v)

        def zrow(i, carry):
            zero_v[i, :] = jnp.zeros((H,), jnp.float32)
            return carry

        lax.fori_loop(0, RPS, zrow, 0)
        pltpu.sync_copy(zero_v, agg_sh.at[pl.ds(sid * RPS, RPS)])
        plsc.subcore_barrier()

        for b in range(NB):
            pltpu.async_copy(y_hbm.at[src_v.at[b]], rows_v.at[b], sem)

        def group(g, carry):
            for b in range(NB):
                j = g * NB + b
                pltpu.make_async_copy(y_hbm.at[src_v.at[j]],
                                      rows_v.at[b], sem).wait()
                pltpu.sync_copy(rows_v.at[b], agg_sh.at[dst_v.at[j]], add=True)
                pltpu.async_copy(y_hbm.at[src_v.at[j + NB]], rows_v.at[b], sem)
            return carry

        lax.fori_loop(0, CPT // NB - 1, group, 0)
        for b in range(NB):
            j = CPT - NB + b
            pltpu.make_async_copy(y_hbm.at[src_v.at[j]],
                                  rows_v.at[b], sem).wait()
            pltpu.sync_copy(rows_v.at[b], agg_sh.at[dst_v.at[j]], add=True)
        plsc.subcore_barrier()
        pltpu.sync_copy(agg_sh.at[pl.ds(sid * RPS, RPS)],
                        out_hbm.at[cid, pl.ds(sid * RPS, RPS)])

    return k(y, src_t, dst_t)


def _tc_proj2(a, w1, w2):
    def body(a_ref, w1_ref, w2_ref, o1_ref, o2_ref):
        av = a_ref[...]
        o1_ref[...] = jnp.dot(av, w1_ref[...], preferred_element_type=jnp.float32)
        o2_ref[...] = jnp.dot(av, w2_ref[...], preferred_element_type=jnp.float32)

    n = a.shape[0]
    h = w1.shape[1]
    return pl.pallas_call(
        body,
        out_shape=[jax.ShapeDtypeStruct((n, h), jnp.float32)] * 2,
    )(a, w1, w2)


def _tc_combine_proj(agg, r, b, w1, w2):
    def body(agg_ref, r_ref, b_ref, w1_ref, w2_ref, o1_ref, o2_ref):
        h = jnp.maximum(agg_ref[0] + agg_ref[1] + r_ref[...] + b_ref[...], 0.0)
        o1_ref[...] = jnp.dot(h, w1_ref[...], preferred_element_type=jnp.float32)
        o2_ref[...] = jnp.dot(h, w2_ref[...], preferred_element_type=jnp.float32)

    n = r.shape[0]
    h = w1.shape[1]
    return pl.pallas_call(
        body,
        out_shape=[jax.ShapeDtypeStruct((n, h), jnp.float32)] * 2,
    )(agg, r, b, w1, w2)


def _tc_combine_out(agg, r, b, wfc_row, bfc):
    def body(agg_ref, r_ref, b_ref, w_ref, bfc_ref, o_ref):
        h = jnp.maximum(agg_ref[0] + agg_ref[1] + r_ref[...] + b_ref[...], 0.0)
        o_ref[...] = (jnp.sum(h * w_ref[...], axis=1, keepdims=True)
                      + bfc_ref[...])

    n = r.shape[0]
    return pl.pallas_call(
        body,
        out_shape=jax.ShapeDtypeStruct((n, 1), jnp.float32),
    )(agg, r, b, wfc_row, bfc)


def kernel(x, edge_index, W1_rel, b1, W1_root, W2_rel, b2, W2_root, W_fc, b_fc):
    e = edge_index.shape[1]
    pad = EPAD - e
    src = jnp.concatenate([edge_index[0], jnp.zeros((pad,), jnp.int32)])
    dst = jnp.concatenate([edge_index[1], jnp.full((pad,), N, jnp.int32)])
    src_t = src.reshape(NW, CPT, CHUNK)
    dst_t = dst.reshape(NW, CPT, CHUNK)

    y1, r1 = _tc_proj2(x, W1_rel, W1_root)
    agg1 = _sc_scatter(y1, src_t, dst_t)[:, :N, :]
    y2, r2 = _tc_combine_proj(agg1, r1, b1.reshape(1, H), W2_rel, W2_root)
    agg2 = _sc_scatter(y2, src_t, dst_t)[:, :N, :]
    out = _tc_combine_out(agg2, r2, b2.reshape(1, H), W_fc.reshape(1, H),
                          b_fc.reshape(1, 1))
    return out


# speedup vs baseline: 1.5271x; 1.5271x over previous
import functools

import jax
import jax.numpy as jnp
from jax import lax
from jax.experimental import pallas as pl
from jax.experimental.pallas import tpu as pltpu
from jax.experimental.pallas import tpu_sc as plsc

N = 10000
D = 128
H = 16
NC = 2
NS = 16
NW = NC * NS
CHUNK = 512
CPT = 20
NB = 4
EPT = CPT * CHUNK
EPAD = NW * EPT
RPS = 632
NPAD = NS * RPS


def _sc_scatter(y, src_t, dst_t):
    mesh = plsc.VectorSubcoreMesh(core_axis_name="c", subcore_axis_name="s")

    @functools.partial(
        pl.kernel,
        out_type=jax.ShapeDtypeStruct((NC, NPAD, H), jnp.float32),
        mesh=mesh,
        scratch_types=[
            pltpu.VMEM((CPT, CHUNK), jnp.int32),
            pltpu.VMEM((CPT, CHUNK), jnp.int32),
            pltpu.VMEM((NB, CHUNK, H), jnp.float32),
            pltpu.VMEM((RPS, H), jnp.float32),
            pltpu.VMEM_SHARED((NPAD, H), jnp.float32),
            pltpu.VMEM_SHARED((NPAD, H), jnp.float32),
            pltpu.SemaphoreType.DMA,
        ],
        compiler_params=pltpu.CompilerParams(use_tc_tiling_on_sc=False),
    )
    def k(y_hbm, src_hbm, dst_hbm, out_hbm, src_v, dst_v, rows_v, zero_v,
          agg_sh, y_sh, sem):
        cid = lax.axis_index("c")
        sid = lax.axis_index("s")
        wid = cid * NS + sid

        pltpu.sync_copy(src_hbm.at[wid], src_v)
        pltpu.sync_copy(dst_hbm.at[wid], dst_v)
        pltpu.sync_copy(y_hbm.at[pl.ds(sid * RPS, RPS)],
                        y_sh.at[pl.ds(sid * RPS, RPS)])

        def zrow(i, carry):
            zero_v[i, :] = jnp.zeros((H,), jnp.float32)
            return carry

        lax.fori_loop(0, RPS, zrow, 0)
        pltpu.sync_copy(zero_v, agg_sh.at[pl.ds(sid * RPS, RPS)])
        plsc.subcore_barrier()

        for b in range(NB):
            pltpu.async_copy(y_sh.at[src_v.at[b]], rows_v.at[b], sem)

        def group(g, carry):
            for b in range(NB):
                j = g * NB + b
                pltpu.make_async_copy(y_sh.at[src_v.at[j]],
                                      rows_v.at[b], sem).wait()
                pltpu.sync_copy(rows_v.at[b], agg_sh.at[dst_v.at[j]], add=True)
                pltpu.async_copy(y_sh.at[src_v.at[j + NB]], rows_v.at[b], sem)
            return carry

        lax.fori_loop(0, CPT // NB - 1, group, 0)
        for b in range(NB):
            j = CPT - NB + b
            pltpu.make_async_copy(y_sh.at[src_v.at[j]],
                                  rows_v.at[b], sem).wait()
            pltpu.sync_copy(rows_v.at[b], agg_sh.at[dst_v.at[j]], add=True)
        plsc.subcore_barrier()
        pltpu.sync_copy(agg_sh.at[pl.ds(sid * RPS, RPS)],
                        out_hbm.at[cid, pl.ds(sid * RPS, RPS)])

    return k(y, src_t, dst_t)


def _tc_proj2(a, w1, w2):
    def body(a_ref, w1_ref, w2_ref, o1_ref, o2_ref):
        av = a_ref[...]
        o1_ref[pl.ds(0, a.shape[0]), :] = jnp.dot(
            av, w1_ref[...], preferred_element_type=jnp.float32)
        o2_ref[...] = jnp.dot(av, w2_ref[...], preferred_element_type=jnp.float32)

    n = a.shape[0]
    h = w1.shape[1]
    return pl.pallas_call(
        body,
        out_shape=[jax.ShapeDtypeStruct((NPAD, h), jnp.float32),
                   jax.ShapeDtypeStruct((n, h), jnp.float32)],
    )(a, w1, w2)


def _tc_combine_proj(agg, r, b, w1, w2):
    def body(agg_ref, r_ref, b_ref, w1_ref, w2_ref, o1_ref, o2_ref):
        h = jnp.maximum(agg_ref[0] + agg_ref[1] + r_ref[...] + b_ref[...], 0.0)
        o1_ref[pl.ds(0, r.shape[0]), :] = jnp.dot(
            h, w1_ref[...], preferred_element_type=jnp.float32)
        o2_ref[...] = jnp.dot(h, w2_ref[...], preferred_element_type=jnp.float32)

    n = r.shape[0]
    h = w1.shape[1]
    return pl.pallas_call(
        body,
        out_shape=[jax.ShapeDtypeStruct((NPAD, h), jnp.float32),
                   jax.ShapeDtypeStruct((n, h), jnp.float32)],
    )(agg, r, b, w1, w2)


def _tc_combine_out(agg, r, b, wfc_row, bfc):
    def body(agg_ref, r_ref, b_ref, w_ref, bfc_ref, o_ref):
        h = jnp.maximum(agg_ref[0] + agg_ref[1] + r_ref[...] + b_ref[...], 0.0)
        o_ref[...] = (jnp.sum(h * w_ref[...], axis=1, keepdims=True)
                      + bfc_ref[...])

    n = r.shape[0]
    return pl.pallas_call(
        body,
        out_shape=jax.ShapeDtypeStruct((n, 1), jnp.float32),
    )(agg, r, b, wfc_row, bfc)


def kernel(x, edge_index, W1_rel, b1, W1_root, W2_rel, b2, W2_root, W_fc, b_fc):
    e = edge_index.shape[1]
    pad = EPAD - e
    src = jnp.concatenate([edge_index[0], jnp.zeros((pad,), jnp.int32)])
    dst = jnp.concatenate([edge_index[1], jnp.full((pad,), N, jnp.int32)])
    src_t = src.reshape(NW, CPT, CHUNK)
    dst_t = dst.reshape(NW, CPT, CHUNK)

    y1, r1 = _tc_proj2(x, W1_rel, W1_root)
    agg1 = _sc_scatter(y1, src_t, dst_t)[:, :N, :]
    y2, r2 = _tc_combine_proj(agg1, r1, b1.reshape(1, H), W2_rel, W2_root)
    agg2 = _sc_scatter(y2, src_t, dst_t)[:, :N, :]
    out = _tc_combine_out(agg2, r2, b2.reshape(1, H), W_fc.reshape(1, H),
                          b_fc.reshape(1, 1))
    return out
